# baseline (speedup 1.0000x reference)
import jax
import jax.numpy as jnp
from jax import lax
from jax.experimental import pallas as pl
from jax.experimental.pallas import tpu as pltpu
from jax.experimental.pallas import tpu_sc as plsc

_NX = 512
_NY = 512
_NBEV = 64
_B = 4
_P = 100000
_D = 524288

_CELLS = _B * _NX * _NY
_NT = 32
_CPT = _CELLS // _NT
_PB = _P // _B
_DB = _D // _B
_CH = 512
_NPC = -(-_PB // _CH)
_NDC = _DB // _CH
_NSENT = 512
_PFW = 80
_HR = 256
_NHRT = _CPT // _HR


def _pipelined(nchunks, start, wait, process):
    start(0, 0)

    def outer(c2, _):
        for b2 in range(2):
            def step(c=c2 * 2 + b2, slot=b2):
                wait(c, slot)

                def prefetch(c=c, slot=slot):
                    start(c + 1, 1 - slot)
                pl.when(c + 1 < nchunks)(prefetch)
                process(c, slot)
            pl.when(c2 * 2 + b2 < nchunks)(step)
        return 0
    lax.fori_loop(0, (nchunks + 1) // 2, outer, 0)


def _sc_scan(yp_hbm, xp_hbm, yd_hbm, xd_hbm, dval_hbm,
             w_hbm, dvp_hbm,
             w_map, dv_map, ybuf, xbuf, vbuf, sem_in0, sem_in1):
    wid = lax.axis_index("s") * 2 + lax.axis_index("c")
    b = wid // 8
    s8 = wid % 8
    cell_lo = s8 * _CPT
    lane = lax.broadcasted_iota(jnp.int32, (16,), 0)
    sems_in = [sem_in0, sem_in1]

    def init_body(i, _):
        w_map[pl.ds(i * 16, 16)] = jnp.full((16,), -1, jnp.int32)
        dv_map[pl.ds(i * 16, 16)] = jnp.zeros((16,), jnp.int32)
        return 0
    lax.fori_loop(0, _CPT // 16, init_body, 0)

    def p_start(c, slot):
        st = b * _PB + c * _CH
        pltpu.async_copy(yp_hbm.at[pl.ds(st, _CH)], ybuf.at[slot], sems_in[slot])
        pltpu.async_copy(xp_hbm.at[pl.ds(st, _CH)], xbuf.at[slot], sems_in[slot])

    def p_wait(c, slot):
        pltpu.make_async_copy(yp_hbm.at[pl.ds(0, _CH)], ybuf.at[slot], sems_in[slot]).wait()
        pltpu.make_async_copy(xp_hbm.at[pl.ds(0, _CH)], xbuf.at[slot], sems_in[slot]).wait()

    def _p_addr(c, slot, g):
        r0 = g * 16
        y = ybuf[slot, pl.ds(r0, 16)]
        x = xbuf[slot, pl.ds(r0, 16)]
        p_loc = c * _CH + r0 + lane
        gm = p_loc < _PB
        rel = y * _NX + x - cell_lo
        inb = gm & (rel >= 0) & (rel < _CPT)
        relc = jnp.clip(rel, 0, _CPT - 1)
        pg = b * _PB + p_loc
        return relc, pg, inb

    def p_process(c, slot):
        def p_group(g2, bad):
            for u in range(2):
                relc, pg, inb = _p_addr(c, slot, g2 * 2 + u)
                plsc.store_scatter(w_map, [relc], pg, mask=inb)
                got = plsc.load_gather(w_map, [relc])
                bad = bad | (inb & (got < pg))
            return bad
        bad = lax.fori_loop(0, _CH // 32, p_group,
                            jnp.zeros((16,), jnp.bool_))

        def p_fix():
            def fix_group(g, _):
                relc, pg, inb = _p_addr(c, slot, g)

                def cond(m):
                    return jnp.any(m)

                def body(m):
                    plsc.store_scatter(w_map, [relc], pg, mask=m)
                    got2 = plsc.load_gather(w_map, [relc])
                    return m & (got2 < pg)

                got = plsc.load_gather(w_map, [relc])
                lax.while_loop(cond, body, inb & (got < pg))
                return 0
            lax.fori_loop(0, _CH // 16, fix_group, 0)
        pl.when(jnp.any(bad))(p_fix)

    _pipelined(_NPC, p_start, p_wait, p_process)

    def d_start(c, slot):
        st = b * _DB + c * _CH
        pltpu.async_copy(yd_hbm.at[pl.ds(st, _CH)], ybuf.at[slot], sems_in[slot])
        pltpu.async_copy(xd_hbm.at[pl.ds(st, _CH)], xbuf.at[slot], sems_in[slot])
        pltpu.async_copy(dval_hbm.at[pl.ds(st, _CH)], vbuf.at[slot], sems_in[slot])

    def d_wait(c, slot):
        pltpu.make_async_copy(yd_hbm.at[pl.ds(0, _CH)], ybuf.at[slot], sems_in[slot]).wait()
        pltpu.make_async_copy(xd_hbm.at[pl.ds(0, _CH)], xbuf.at[slot], sems_in[slot]).wait()
        pltpu.make_async_copy(dval_hbm.at[pl.ds(0, _CH)], vbuf.at[slot], sems_in[slot]).wait()

    def _d_addr(c, slot, g):
        r0 = g * 16
        y = ybuf[slot, pl.ds(r0, 16)]
        x = xbuf[slot, pl.ds(r0, 16)]
        d_loc = c * _CH + r0 + lane
        vi = vbuf[slot, pl.ds(r0, 16)].astype(jnp.int32)
        packed = (d_loc << 4) | vi
        rel = y * _NX + x - cell_lo
        inb = (rel >= 0) & (rel < _CPT)
        relc = jnp.clip(rel, 0, _CPT - 1)
        return relc, packed, inb

    def d_process(c, slot):
        def d_group(g2, bad):
            for u in range(2):
                relc, packed, inb = _d_addr(c, slot, g2 * 2 + u)
                plsc.store_scatter(dv_map, [relc], packed, mask=inb)
                got = plsc.load_gather(dv_map, [relc])
                bad = bad | (inb & (got < packed))
            return bad
        bad = lax.fori_loop(0, _CH // 32, d_group,
                            jnp.zeros((16,), jnp.bool_))

        def d_fix():
            def fix_group(g, _):
                relc, packed, inb = _d_addr(c, slot, g)

                def cond(m):
                    return jnp.any(m)

                def body(m):
                    plsc.store_scatter(dv_map, [relc], packed, mask=m)
                    got2 = plsc.load_gather(dv_map, [relc])
                    return m & (got2 < packed)

                got = plsc.load_gather(dv_map, [relc])
                lax.while_loop(cond, body, inb & (got < packed))
                return 0
            lax.fori_loop(0, _CH // 16, fix_group, 0)
        pl.when(jnp.any(bad))(d_fix)

    _pipelined(_NDC, d_start, d_wait, d_process)

    def fix_body(i, _):
        w = w_map[pl.ds(i * 16, 16)]
        sent = _P + ((i * 16 + lane) & (_NSENT - 1))
        w_map[pl.ds(i * 16, 16)] = jnp.where(w < 0, sent, w)
        return 0
    lax.fori_loop(0, _CPT // 16, fix_body, 0)

    pltpu.sync_copy(w_map, w_hbm.at[pl.ds(wid * _CPT, _CPT)])
    pltpu.sync_copy(dv_map, dvp_hbm.at[pl.ds(wid * _CPT, _CPT)])


def _sc_produce(w_hbm, dvp_hbm, pf_hbm,
                feats_hbm, seg_hbm, pm_hbm, oh_hbm,
                wrow, dvrow, rows, tr, oh,
                sem_w0, sem_w1, sem_g0, sem_g1, sem_o0, sem_o1):
    wid = lax.axis_index("s") * 2 + lax.axis_index("c")
    lane = lax.broadcasted_iota(jnp.int32, (16,), 0)
    sems_w = [sem_w0, sem_w1]
    sems_g = [sem_g0, sem_g1]
    sems_o = [sem_o0, sem_o1]
    hr0 = wid * _NHRT

    def stage(j, s):
        r2 = (hr0 + j) * 2
        pltpu.async_copy(w_hbm.at[pl.ds(r2, 2)], wrow.at[s], sems_w[s])
        pltpu.async_copy(dvp_hbm.at[pl.ds(r2, 2)], dvrow.at[s], sems_w[s])

    def wait_stage(s):
        pltpu.make_async_copy(w_hbm.at[pl.ds(0, 2)], wrow.at[s], sems_w[s]).wait()
        pltpu.make_async_copy(dvp_hbm.at[pl.ds(0, 2)], dvrow.at[s], sems_w[s]).wait()

    def fire_gather(j, s):
        for k in range(2):
            pltpu.async_copy(pf_hbm.at[wrow.at[s, k]],
                             rows.at[s, pl.ds(k * 128, 128)], sems_g[s])

    def wait_gather(s):
        for k in range(2):
            pltpu.make_async_copy(pf_hbm.at[wrow.at[s, k]],
                                  rows.at[s, pl.ds(k * 128, 128)],
                                  sems_g[s]).wait()

    def _out_slices(j, s):
        hr = hr0 + j
        b = hr // 1024
        col = (hr % 1024) * _HR
        return [
            (tr.at[s, pl.ds(0, _NBEV)],
             feats_hbm.at[pl.ds(b * _NBEV, _NBEV), pl.ds(col, _HR)]),
            (tr.at[s, pl.ds(_NBEV, 1)],
             seg_hbm.at[pl.ds(b, 1), pl.ds(col, _HR)]),
            (tr.at[s, pl.ds(_NBEV + 1, 3)],
             pm_hbm.at[pl.ds(b * 3, 3), pl.ds(col, _HR)]),
            (oh.at[s],
             oh_hbm.at[pl.ds(b * 16, 16), pl.ds(col, _HR)]),
        ]

    def fire_outs(j, s):
        for src, dst in _out_slices(j, s):
            pltpu.async_copy(src, dst, sems_o[s])

    def wait_outs(j, s):
        for src, dst in _out_slices(j, s):
            pltpu.make_async_copy(src, dst, sems_o[s]).wait()

    def compute(j, s):
        def tgroup(g, _):
            ridx = g * 16 + lane
            for c in range(68):
                cc = jnp.full((16,), c, jnp.int32)
                tr[s, c, pl.ds(g * 16, 16)] = plsc.load_gather(
                    rows.at[s], [ridx, cc])
            return 0
        lax.fori_loop(0, _HR // 16, tgroup, 0)

        def sgroup(g, _):
            sp = tr[s, _NBEV, pl.ds(g * 16, 16)]
            dq = dvrow[s, g // 8, pl.ds((g % 8) * 16, 16)]
            dvf = (dq & 15).astype(jnp.float32)
            seg = jnp.where(sp == 0.0, dvf, sp)
            tr[s, _NBEV, pl.ds(g * 16, 16)] = seg
            si = seg.astype(jnp.int32)
            for cls in range(16):
                oh[s, cls, pl.ds(g * 16, 16)] = (si == cls).astype(jnp.float32)
            return 0
        lax.fori_loop(0, _HR // 16, sgroup, 0)

    stage(0, 0)
    stage(1, 1)
    wait_stage(0)
    fire_gather(0, 0)

    def outer(j2, _):
        for u in range(2):
            def step(j=j2 * 2 + u, s=u):
                def nxt(j=j, s=s):
                    wait_stage(1 - s)
                    fire_gather(j + 1, 1 - s)
                pl.when(j + 1 < _NHRT)(nxt)
                wait_gather(s)

                def drain(j=j, s=s):
                    wait_outs(j - 2, s)
                pl.when(j >= 2)(drain)
                compute(j, s)

                def restage(j=j, s=s):
                    stage(j + 2, s)
                pl.when(j + 2 < _NHRT)(restage)
                fire_outs(j, s)
            pl.when(j2 * 2 + u < _NHRT)(step)
        return 0
    lax.fori_loop(0, _NHRT // 2, outer, 0)

    wait_outs(_NHRT - 2, 0)
    wait_outs(_NHRT - 1, 1)


def kernel(pillar_features, voxel_coords, pillar_seg_gt, pillar_dense_gt, dense_pillar_coords, points_mean):
    pm = points_mean.reshape(_P, 3)
    pf_ext = jnp.concatenate(
        [pillar_features, pillar_seg_gt, pm,
         jnp.zeros((_P, _PFW - 68), jnp.float32)], axis=1)
    pf_ext = jnp.concatenate([pf_ext, jnp.zeros((_NSENT, _PFW), jnp.float32)],
                             axis=0)

    pad = _NPC * _CH - _PB
    yp = jnp.pad(voxel_coords[:, 2], (0, pad + 8))
    xp = jnp.pad(voxel_coords[:, 3], (0, pad + 8))
    yd = dense_pillar_coords[:, 2]
    xd = dense_pillar_coords[:, 3]
    dval = pillar_dense_gt.reshape(-1)

    mesh = plsc.VectorSubcoreMesh(core_axis_name="c", subcore_axis_name="s")
    sc_params = pltpu.CompilerParams(
        needs_layout_passes=False, use_tc_tiling_on_sc=False)

    w_flat, dvp_flat = pl.kernel(
        _sc_scan,
        mesh=mesh,
        compiler_params=sc_params,
        out_type=[
            jax.ShapeDtypeStruct((_CELLS,), jnp.int32),
            jax.ShapeDtypeStruct((_CELLS,), jnp.int32),
        ],
        scratch_types=[
            pltpu.VMEM((_CPT,), jnp.int32),
            pltpu.VMEM((_CPT,), jnp.int32),
            pltpu.VMEM((2, _CH), jnp.int32),
            pltpu.VMEM((2, _CH), jnp.int32),
            pltpu.VMEM((2, _CH), jnp.float32),
            pltpu.SemaphoreType.DMA,
            pltpu.SemaphoreType.DMA,
        ],
    )(yp, xp, yd, xd, dval)

    w2 = w_flat.reshape(_CELLS // 128, 128)
    dvp2 = dvp_flat.reshape(_CELLS // 128, 128)

    feats2, seg2, pm2, oh2 = pl.kernel(
        _sc_produce,
        mesh=mesh,
        compiler_params=sc_params,
        out_type=[
            jax.ShapeDtypeStruct((_B * _NBEV, _NY * _NX), jnp.float32),
            jax.ShapeDtypeStruct((_B, _NY * _NX), jnp.float32),
            jax.ShapeDtypeStruct((_B * 3, _NY * _NX), jnp.float32),
            jax.ShapeDtypeStruct((_B * 16, _NY * _NX), jnp.float32),
        ],
        scratch_types=[
            pltpu.VMEM((2, 2, 128), jnp.int32),
            pltpu.VMEM((2, 2, 128), jnp.int32),
            pltpu.VMEM((2, _HR, _PFW), jnp.float32),
            pltpu.VMEM((2, _PFW, _HR), jnp.float32),
            pltpu.VMEM((2, 16, _HR), jnp.float32),
            pltpu.SemaphoreType.DMA,
            pltpu.SemaphoreType.DMA,
            pltpu.SemaphoreType.DMA,
            pltpu.SemaphoreType.DMA,
            pltpu.SemaphoreType.DMA,
            pltpu.SemaphoreType.DMA,
        ],
    )(w2, dvp2, pf_ext)

    feats = feats2.reshape(_B, _NBEV, _NY, _NX)
    seg = seg2.reshape(_B, 1, _NY, _NX)
    pm3 = pm2.reshape(_B, 3, _NY, _NX)
    onehot = oh2.reshape(_B, 16, _NY, _NX)
    return feats, seg, pm3, onehot


# speedup vs baseline: 1.3376x; 1.3376x over previous
import jax
import jax.numpy as jnp
from jax import lax
from jax.experimental import pallas as pl
from jax.experimental.pallas import tpu as pltpu
from jax.experimental.pallas import tpu_sc as plsc

_NX = 512
_NY = 512
_NBEV = 64
_B = 4
_P = 100000
_D = 524288

_CELLS = _B * _NX * _NY
_NT = 32
_CPT = _CELLS // _NT
_PB = _P // _B
_DB = _D // _B
_CH = 512
_NPC = -(-_PB // _CH)
_NDC = _DB // _CH
_NSENT = 512
_PFW = 80
_HR = 256
_NHRT = _CPT // _HR
_TPITCH = 257


def _pipelined(nchunks, start, wait, process):
    start(0, 0)

    def outer(c2, _):
        for b2 in range(2):
            def step(c=c2 * 2 + b2, slot=b2):
                wait(c, slot)

                def prefetch(c=c, slot=slot):
                    start(c + 1, 1 - slot)
                pl.when(c + 1 < nchunks)(prefetch)
                process(c, slot)
            pl.when(c2 * 2 + b2 < nchunks)(step)
        return 0
    lax.fori_loop(0, (nchunks + 1) // 2, outer, 0)


def _sc_scan(yp_hbm, xp_hbm, yd_hbm, xd_hbm, dval_hbm,
             w_hbm, dvp_hbm,
             w_map, dv_map, ybuf, xbuf, vbuf, sem_in0, sem_in1):
    wid = lax.axis_index("s") * 2 + lax.axis_index("c")
    b = wid // 8
    s8 = wid % 8
    cell_lo = s8 * _CPT
    lane = lax.broadcasted_iota(jnp.int32, (16,), 0)
    sems_in = [sem_in0, sem_in1]

    def init_body(i, _):
        w_map[pl.ds(i * 16, 16)] = jnp.full((16,), -1, jnp.int32)
        dv_map[pl.ds(i * 16, 16)] = jnp.zeros((16,), jnp.int32)
        return 0
    lax.fori_loop(0, _CPT // 16, init_body, 0)

    def p_start(c, slot):
        st = b * _PB + c * _CH
        pltpu.async_copy(yp_hbm.at[pl.ds(st, _CH)], ybuf.at[slot], sems_in[slot])
        pltpu.async_copy(xp_hbm.at[pl.ds(st, _CH)], xbuf.at[slot], sems_in[slot])

    def p_wait(c, slot):
        pltpu.make_async_copy(yp_hbm.at[pl.ds(0, _CH)], ybuf.at[slot], sems_in[slot]).wait()
        pltpu.make_async_copy(xp_hbm.at[pl.ds(0, _CH)], xbuf.at[slot], sems_in[slot]).wait()

    def _p_addr(c, slot, g):
        r0 = g * 16
        y = ybuf[slot, pl.ds(r0, 16)]
        x = xbuf[slot, pl.ds(r0, 16)]
        p_loc = c * _CH + r0 + lane
        gm = p_loc < _PB
        rel = y * _NX + x - cell_lo
        inb = gm & (rel >= 0) & (rel < _CPT)
        relc = jnp.clip(rel, 0, _CPT - 1)
        pg = b * _PB + p_loc
        return relc, pg, inb

    def p_process(c, slot):
        def p_group(g2, bad):
            for u in range(2):
                relc, pg, inb = _p_addr(c, slot, g2 * 2 + u)
                plsc.store_scatter(w_map, [relc], pg, mask=inb)
                got = plsc.load_gather(w_map, [relc])
                bad = bad | (inb & (got < pg))
            return bad
        bad = lax.fori_loop(0, _CH // 32, p_group,
                            jnp.zeros((16,), jnp.bool_))

        def p_fix():
            def fix_group(g, _):
                relc, pg, inb = _p_addr(c, slot, g)

                def cond(m):
                    return jnp.any(m)

                def body(m):
                    plsc.store_scatter(w_map, [relc], pg, mask=m)
                    got2 = plsc.load_gather(w_map, [relc])
                    return m & (got2 < pg)

                got = plsc.load_gather(w_map, [relc])
                lax.while_loop(cond, body, inb & (got < pg))
                return 0
            lax.fori_loop(0, _CH // 16, fix_group, 0)
        pl.when(jnp.any(bad))(p_fix)

    _pipelined(_NPC, p_start, p_wait, p_process)

    def d_start(c, slot):
        st = b * _DB + c * _CH
        pltpu.async_copy(yd_hbm.at[pl.ds(st, _CH)], ybuf.at[slot], sems_in[slot])
        pltpu.async_copy(xd_hbm.at[pl.ds(st, _CH)], xbuf.at[slot], sems_in[slot])
        pltpu.async_copy(dval_hbm.at[pl.ds(st, _CH)], vbuf.at[slot], sems_in[slot])

    def d_wait(c, slot):
        pltpu.make_async_copy(yd_hbm.at[pl.ds(0, _CH)], ybuf.at[slot], sems_in[slot]).wait()
        pltpu.make_async_copy(xd_hbm.at[pl.ds(0, _CH)], xbuf.at[slot], sems_in[slot]).wait()
        pltpu.make_async_copy(dval_hbm.at[pl.ds(0, _CH)], vbuf.at[slot], sems_in[slot]).wait()

    def _d_addr(c, slot, g):
        r0 = g * 16
        y = ybuf[slot, pl.ds(r0, 16)]
        x = xbuf[slot, pl.ds(r0, 16)]
        d_loc = c * _CH + r0 + lane
        vi = vbuf[slot, pl.ds(r0, 16)].astype(jnp.int32)
        packed = (d_loc << 4) | vi
        rel = y * _NX + x - cell_lo
        inb = (rel >= 0) & (rel < _CPT)
        relc = jnp.clip(rel, 0, _CPT - 1)
        return relc, packed, inb

    def d_process(c, slot):
        def d_group(g2, bad):
            for u in range(2):
                relc, packed, inb = _d_addr(c, slot, g2 * 2 + u)
                plsc.store_scatter(dv_map, [relc], packed, mask=inb)
                got = plsc.load_gather(dv_map, [relc])
                bad = bad | (inb & (got < packed))
            return bad
        bad = lax.fori_loop(0, _CH // 32, d_group,
                            jnp.zeros((16,), jnp.bool_))

        def d_fix():
            def fix_group(g, _):
                relc, packed, inb = _d_addr(c, slot, g)

                def cond(m):
                    return jnp.any(m)

                def body(m):
                    plsc.store_scatter(dv_map, [relc], packed, mask=m)
                    got2 = plsc.load_gather(dv_map, [relc])
                    return m & (got2 < packed)

                got = plsc.load_gather(dv_map, [relc])
                lax.while_loop(cond, body, inb & (got < packed))
                return 0
            lax.fori_loop(0, _CH // 16, fix_group, 0)
        pl.when(jnp.any(bad))(d_fix)

    _pipelined(_NDC, d_start, d_wait, d_process)

    def fix_body(i, _):
        w = w_map[pl.ds(i * 16, 16)]
        sent = _P + ((i * 16 + lane) & (_NSENT - 1))
        w_map[pl.ds(i * 16, 16)] = jnp.where(w < 0, sent, w)
        return 0
    lax.fori_loop(0, _CPT // 16, fix_body, 0)

    pltpu.sync_copy(w_map, w_hbm.at[pl.ds(wid * _CPT, _CPT)])
    pltpu.sync_copy(dv_map, dvp_hbm.at[pl.ds(wid * _CPT, _CPT)])


def _sc_produce(w_hbm, dvp_hbm, pf_hbm,
                feats_hbm, seg_hbm, pm_hbm, oh_hbm,
                wrow, dvrow, rows, tr, oh,
                sem_w0, sem_w1, sem_g0, sem_g1, sem_o0, sem_o1):
    wid = lax.axis_index("s") * 2 + lax.axis_index("c")
    lane = lax.broadcasted_iota(jnp.int32, (16,), 0)
    sems_w = [sem_w0, sem_w1]
    sems_g = [sem_g0, sem_g1]
    sems_o = [sem_o0, sem_o1]
    hr0 = wid * _NHRT

    def stage(j, s):
        r2 = (hr0 + j) * 2
        pltpu.async_copy(w_hbm.at[pl.ds(r2, 2)], wrow.at[s], sems_w[s])
        pltpu.async_copy(dvp_hbm.at[pl.ds(r2, 2)], dvrow.at[s], sems_w[s])

    def wait_stage(s):
        pltpu.make_async_copy(w_hbm.at[pl.ds(0, 2)], wrow.at[s], sems_w[s]).wait()
        pltpu.make_async_copy(dvp_hbm.at[pl.ds(0, 2)], dvrow.at[s], sems_w[s]).wait()

    def fire_gather(j, s):
        for k in range(2):
            pltpu.async_copy(pf_hbm.at[wrow.at[s, k]],
                             rows.at[s, pl.ds(k * 128, 128)], sems_g[s])

    def wait_gather(s):
        for k in range(2):
            pltpu.make_async_copy(pf_hbm.at[wrow.at[s, k]],
                                  rows.at[s, pl.ds(k * 128, 128)],
                                  sems_g[s]).wait()

    def _out_slices(j, s):
        hr = hr0 + j
        b = hr // 1024
        col = (hr % 1024) * _HR
        return [
            (tr.at[s, pl.ds(0, _NBEV), pl.ds(0, _HR)],
             feats_hbm.at[pl.ds(b * _NBEV, _NBEV), pl.ds(col, _HR)]),
            (tr.at[s, pl.ds(_NBEV, 1), pl.ds(0, _HR)],
             seg_hbm.at[pl.ds(b, 1), pl.ds(col, _HR)]),
            (tr.at[s, pl.ds(_NBEV + 1, 3), pl.ds(0, _HR)],
             pm_hbm.at[pl.ds(b * 3, 3), pl.ds(col, _HR)]),
            (oh.at[s],
             oh_hbm.at[pl.ds(b * 16, 16), pl.ds(col, _HR)]),
        ]

    def fire_outs(j, s):
        for src, dst in _out_slices(j, s):
            pltpu.async_copy(src, dst, sems_o[s])

    def wait_outs(j, s):
        for src, dst in _out_slices(j, s):
            pltpu.make_async_copy(src, dst, sems_o[s]).wait()

    def compute(j, s):
        def tgroup(r, _):
            rr = jnp.full((16,), r, jnp.int32)
            for cb in range(5):
                c0 = cb * 16
                cvec = c0 + lane
                v = rows[s, r, pl.ds(c0, 16)]
                if cb == 4:
                    plsc.store_scatter(tr.at[s], [jnp.minimum(cvec, 67), rr],
                                       v, mask=cvec < 68)
                else:
                    plsc.store_scatter(tr.at[s], [cvec, rr], v)
            return 0
        lax.fori_loop(0, _HR, tgroup, 0)

        def sgroup(g, _):
            sp = tr[s, _NBEV, pl.ds(g * 16, 16)]
            dq = dvrow[s, g // 8, pl.ds((g % 8) * 16, 16)]
            dvf = (dq & 15).astype(jnp.float32)
            seg = jnp.where(sp == 0.0, dvf, sp)
            tr[s, _NBEV, pl.ds(g * 16, 16)] = seg
            si = seg.astype(jnp.int32)
            for cls in range(16):
                oh[s, cls, pl.ds(g * 16, 16)] = (si == cls).astype(jnp.float32)
            return 0
        lax.fori_loop(0, _HR // 16, sgroup, 0)

    stage(0, 0)
    stage(1, 1)
    wait_stage(0)
    fire_gather(0, 0)

    def outer(j2, _):
        for u in range(2):
            def step(j=j2 * 2 + u, s=u):
                def nxt(j=j, s=s):
                    wait_stage(1 - s)
                    fire_gather(j + 1, 1 - s)
                pl.when(j + 1 < _NHRT)(nxt)
                wait_gather(s)

                def drain(j=j, s=s):
                    wait_outs(j - 2, s)
                pl.when(j >= 2)(drain)
                compute(j, s)

                def restage(j=j, s=s):
                    stage(j + 2, s)
                pl.when(j + 2 < _NHRT)(restage)
                fire_outs(j, s)
            pl.when(j2 * 2 + u < _NHRT)(step)
        return 0
    lax.fori_loop(0, _NHRT // 2, outer, 0)

    wait_outs(_NHRT - 2, 0)
    wait_outs(_NHRT - 1, 1)


def kernel(pillar_features, voxel_coords, pillar_seg_gt, pillar_dense_gt, dense_pillar_coords, points_mean):
    pm = points_mean.reshape(_P, 3)
    pf_ext = jnp.concatenate(
        [pillar_features, pillar_seg_gt, pm,
         jnp.zeros((_P, _PFW - 68), jnp.float32)], axis=1)
    pf_ext = jnp.concatenate([pf_ext, jnp.zeros((_NSENT, _PFW), jnp.float32)],
                             axis=0)

    pad = _NPC * _CH - _PB
    yp = jnp.pad(voxel_coords[:, 2], (0, pad + 8))
    xp = jnp.pad(voxel_coords[:, 3], (0, pad + 8))
    yd = dense_pillar_coords[:, 2]
    xd = dense_pillar_coords[:, 3]
    dval = pillar_dense_gt.reshape(-1)

    mesh = plsc.VectorSubcoreMesh(core_axis_name="c", subcore_axis_name="s")
    sc_params = pltpu.CompilerParams(
        needs_layout_passes=False, use_tc_tiling_on_sc=False)

    w_flat, dvp_flat = pl.kernel(
        _sc_scan,
        mesh=mesh,
        compiler_params=sc_params,
        out_type=[
            jax.ShapeDtypeStruct((_CELLS,), jnp.int32),
            jax.ShapeDtypeStruct((_CELLS,), jnp.int32),
        ],
        scratch_types=[
            pltpu.VMEM((_CPT,), jnp.int32),
            pltpu.VMEM((_CPT,), jnp.int32),
            pltpu.VMEM((2, _CH), jnp.int32),
            pltpu.VMEM((2, _CH), jnp.int32),
            pltpu.VMEM((2, _CH), jnp.float32),
            pltpu.SemaphoreType.DMA,
            pltpu.SemaphoreType.DMA,
        ],
    )(yp, xp, yd, xd, dval)

    w2 = w_flat.reshape(_CELLS // 128, 128)
    dvp2 = dvp_flat.reshape(_CELLS // 128, 128)

    feats2, seg2, pm2, oh2 = pl.kernel(
        _sc_produce,
        mesh=mesh,
        compiler_params=sc_params,
        out_type=[
            jax.ShapeDtypeStruct((_B * _NBEV, _NY * _NX), jnp.float32),
            jax.ShapeDtypeStruct((_B, _NY * _NX), jnp.float32),
            jax.ShapeDtypeStruct((_B * 3, _NY * _NX), jnp.float32),
            jax.ShapeDtypeStruct((_B * 16, _NY * _NX), jnp.float32),
        ],
        scratch_types=[
            pltpu.VMEM((2, 2, 128), jnp.int32),
            pltpu.VMEM((2, 2, 128), jnp.int32),
            pltpu.VMEM((2, _HR, _PFW), jnp.float32),
            pltpu.VMEM((2, 68, _TPITCH), jnp.float32),
            pltpu.VMEM((2, 16, _HR), jnp.float32),
            pltpu.SemaphoreType.DMA,
            pltpu.SemaphoreType.DMA,
            pltpu.SemaphoreType.DMA,
            pltpu.SemaphoreType.DMA,
            pltpu.SemaphoreType.DMA,
            pltpu.SemaphoreType.DMA,
        ],
    )(w2, dvp2, pf_ext)

    feats = feats2.reshape(_B, _NBEV, _NY, _NX)
    seg = seg2.reshape(_B, 1, _NY, _NX)
    pm3 = pm2.reshape(_B, 3, _NY, _NX)
    onehot = oh2.reshape(_B, 16, _NY, _NX)
    return feats, seg, pm3, onehot


# speedup vs baseline: 1.3591x; 1.0161x over previous
import jax
import jax.numpy as jnp
from jax import lax
from jax.experimental import pallas as pl
from jax.experimental.pallas import tpu as pltpu
from jax.experimental.pallas import tpu_sc as plsc

_NX = 512
_NY = 512
_NBEV = 64
_B = 4
_P = 100000
_D = 524288

_CELLS = _B * _NX * _NY
_NT = 32
_CPT = _CELLS // _NT
_PB = _P // _B
_DB = _D // _B
_CH = 512
_NPC = -(-_PB // _CH)
_NDC = _DB // _CH
_NSENT = 512
_PFW = 80
_HR = 256
_NHRT = _CPT // _HR
_TPITCH = 257


def _pipelined(nchunks, start, wait, process):
    start(0, 0)

    def outer(c2, _):
        for b2 in range(2):
            def step(c=c2 * 2 + b2, slot=b2):
                wait(c, slot)

                def prefetch(c=c, slot=slot):
                    start(c + 1, 1 - slot)
                pl.when(c + 1 < nchunks)(prefetch)
                process(c, slot)
            pl.when(c2 * 2 + b2 < nchunks)(step)
        return 0
    lax.fori_loop(0, (nchunks + 1) // 2, outer, 0)


def _sc_scan(yp_hbm, xp_hbm, yd_hbm, xd_hbm, dval_hbm,
             w_hbm, dvp_hbm,
             w_map, dv_map, ybuf, xbuf, vbuf, sem_in0, sem_in1):
    wid = lax.axis_index("s") * 2 + lax.axis_index("c")
    b = wid // 8
    s8 = wid % 8
    cell_lo = s8 * _CPT
    lane = lax.broadcasted_iota(jnp.int32, (16,), 0)
    sems_in = [sem_in0, sem_in1]

    def init_body(i, _):
        w_map[pl.ds(i * 16, 16)] = jnp.full((16,), -1, jnp.int32)
        dv_map[pl.ds(i * 16, 16)] = jnp.zeros((16,), jnp.int32)
        return 0
    lax.fori_loop(0, _CPT // 16, init_body, 0)

    def p_start(c, slot):
        st = b * _PB + c * _CH
        pltpu.async_copy(yp_hbm.at[pl.ds(st, _CH)], ybuf.at[slot], sems_in[slot])
        pltpu.async_copy(xp_hbm.at[pl.ds(st, _CH)], xbuf.at[slot], sems_in[slot])

    def p_wait(c, slot):
        pltpu.make_async_copy(yp_hbm.at[pl.ds(0, _CH)], ybuf.at[slot], sems_in[slot]).wait()
        pltpu.make_async_copy(xp_hbm.at[pl.ds(0, _CH)], xbuf.at[slot], sems_in[slot]).wait()

    def _p_addr(c, slot, g):
        r0 = g * 16
        y = ybuf[slot, pl.ds(r0, 16)]
        x = xbuf[slot, pl.ds(r0, 16)]
        p_loc = c * _CH + r0 + lane
        gm = p_loc < _PB
        rel = y * _NX + x - cell_lo
        inb = gm & (rel >= 0) & (rel < _CPT)
        relc = jnp.clip(rel, 0, _CPT - 1)
        pg = b * _PB + p_loc
        return relc, pg, inb

    def p_process(c, slot):
        def p_group(g2, bad):
            for u in range(2):
                relc, pg, inb = _p_addr(c, slot, g2 * 2 + u)
                plsc.store_scatter(w_map, [relc], pg, mask=inb)
                got = plsc.load_gather(w_map, [relc])
                bad = bad | (inb & (got < pg))
            return bad
        bad = lax.fori_loop(0, _CH // 32, p_group,
                            jnp.zeros((16,), jnp.bool_))

        def p_fix():
            def fix_group(g, _):
                relc, pg, inb = _p_addr(c, slot, g)

                def cond(m):
                    return jnp.any(m)

                def body(m):
                    plsc.store_scatter(w_map, [relc], pg, mask=m)
                    got2 = plsc.load_gather(w_map, [relc])
                    return m & (got2 < pg)

                got = plsc.load_gather(w_map, [relc])
                lax.while_loop(cond, body, inb & (got < pg))
                return 0
            lax.fori_loop(0, _CH // 16, fix_group, 0)
        pl.when(jnp.any(bad))(p_fix)

    _pipelined(_NPC, p_start, p_wait, p_process)

    def d_start(c, slot):
        st = b * _DB + c * _CH
        pltpu.async_copy(yd_hbm.at[pl.ds(st, _CH)], ybuf.at[slot], sems_in[slot])
        pltpu.async_copy(xd_hbm.at[pl.ds(st, _CH)], xbuf.at[slot], sems_in[slot])
        pltpu.async_copy(dval_hbm.at[pl.ds(st, _CH)], vbuf.at[slot], sems_in[slot])

    def d_wait(c, slot):
        pltpu.make_async_copy(yd_hbm.at[pl.ds(0, _CH)], ybuf.at[slot], sems_in[slot]).wait()
        pltpu.make_async_copy(xd_hbm.at[pl.ds(0, _CH)], xbuf.at[slot], sems_in[slot]).wait()
        pltpu.make_async_copy(dval_hbm.at[pl.ds(0, _CH)], vbuf.at[slot], sems_in[slot]).wait()

    def _d_addr(c, slot, g):
        r0 = g * 16
        y = ybuf[slot, pl.ds(r0, 16)]
        x = xbuf[slot, pl.ds(r0, 16)]
        d_loc = c * _CH + r0 + lane
        vi = vbuf[slot, pl.ds(r0, 16)].astype(jnp.int32)
        packed = (d_loc << 4) | vi
        rel = y * _NX + x - cell_lo
        inb = (rel >= 0) & (rel < _CPT)
        relc = jnp.clip(rel, 0, _CPT - 1)
        return relc, packed, inb

    def d_process(c, slot):
        def d_group(g2, bad):
            for u in range(2):
                relc, packed, inb = _d_addr(c, slot, g2 * 2 + u)
                plsc.store_scatter(dv_map, [relc], packed, mask=inb)
                got = plsc.load_gather(dv_map, [relc])
                bad = bad | (inb & (got < packed))
            return bad
        bad = lax.fori_loop(0, _CH // 32, d_group,
                            jnp.zeros((16,), jnp.bool_))

        def d_fix():
            def fix_group(g, _):
                relc, packed, inb = _d_addr(c, slot, g)

                def cond(m):
                    return jnp.any(m)

                def body(m):
                    plsc.store_scatter(dv_map, [relc], packed, mask=m)
                    got2 = plsc.load_gather(dv_map, [relc])
                    return m & (got2 < packed)

                got = plsc.load_gather(dv_map, [relc])
                lax.while_loop(cond, body, inb & (got < packed))
                return 0
            lax.fori_loop(0, _CH // 16, fix_group, 0)
        pl.when(jnp.any(bad))(d_fix)

    _pipelined(_NDC, d_start, d_wait, d_process)

    def fix_body(i, _):
        w = w_map[pl.ds(i * 16, 16)]
        sent = _P + ((i * 16 + lane) & (_NSENT - 1))
        w_map[pl.ds(i * 16, 16)] = jnp.where(w < 0, sent, w)
        return 0
    lax.fori_loop(0, _CPT // 16, fix_body, 0)

    pltpu.sync_copy(w_map, w_hbm.at[pl.ds(wid * _CPT, _CPT)])
    pltpu.sync_copy(dv_map, dvp_hbm.at[pl.ds(wid * _CPT, _CPT)])


def _sc_produce(w_hbm, dvp_hbm, pf_hbm,
                feats_hbm, seg_hbm, pm_hbm, oh_hbm,
                wrow, dvrow, rows, tr, oh,
                sem_w0, sem_w1, sem_g0, sem_g1, sem_o0, sem_o1):
    wid = lax.axis_index("s") * 2 + lax.axis_index("c")
    lane = lax.broadcasted_iota(jnp.int32, (16,), 0)
    sems_w = [sem_w0, sem_w1]
    sems_g = [sem_g0, sem_g1]
    sems_o = [sem_o0, sem_o1]
    hr0 = wid * _NHRT

    def stage(j, s):
        r2 = (hr0 + j) * 2
        pltpu.async_copy(w_hbm.at[pl.ds(r2, 2)], wrow.at[s], sems_w[s])
        pltpu.async_copy(dvp_hbm.at[pl.ds(r2, 2)], dvrow.at[s], sems_w[s])

    def wait_stage(s):
        pltpu.make_async_copy(w_hbm.at[pl.ds(0, 2)], wrow.at[s], sems_w[s]).wait()
        pltpu.make_async_copy(dvp_hbm.at[pl.ds(0, 2)], dvrow.at[s], sems_w[s]).wait()

    def fire_gather(j, s):
        for k in range(2):
            pltpu.async_copy(pf_hbm.at[wrow.at[s, k]],
                             rows.at[s, pl.ds(k * 128, 128)], sems_g[s])

    def wait_gather(s):
        for k in range(2):
            pltpu.make_async_copy(pf_hbm.at[wrow.at[s, k]],
                                  rows.at[s, pl.ds(k * 128, 128)],
                                  sems_g[s]).wait()

    def _out_slices(j, s):
        hr = hr0 + j
        b = hr // 1024
        col = (hr % 1024) * _HR
        return [
            (tr.at[s, pl.ds(0, _NBEV), pl.ds(0, _HR)],
             feats_hbm.at[pl.ds(b * _NBEV, _NBEV), pl.ds(col, _HR)]),
            (tr.at[s, pl.ds(_NBEV, 1), pl.ds(0, _HR)],
             seg_hbm.at[pl.ds(b, 1), pl.ds(col, _HR)]),
            (tr.at[s, pl.ds(_NBEV + 1, 3), pl.ds(0, _HR)],
             pm_hbm.at[pl.ds(b * 3, 3), pl.ds(col, _HR)]),
            (oh.at[s],
             oh_hbm.at[pl.ds(b * 16, 16), pl.ds(col, _HR)]),
        ]

    def fire_outs(j, s):
        for src, dst in _out_slices(j, s):
            pltpu.async_copy(src, dst, sems_o[s])

    def wait_outs(j, s):
        for src, dst in _out_slices(j, s):
            pltpu.make_async_copy(src, dst, sems_o[s]).wait()

    def compute(j, s):
        def tgroup(r4, _):
            for u in range(4):
                r = r4 * 4 + u
                rr = jnp.full((16,), r, jnp.int32)
                for cb in range(5):
                    c0 = cb * 16
                    cvec = c0 + lane
                    v = rows[s, r, pl.ds(c0, 16)]
                    if cb == 4:
                        plsc.store_scatter(tr.at[s], [jnp.minimum(cvec, 67), rr],
                                           v, mask=cvec < 68)
                    else:
                        plsc.store_scatter(tr.at[s], [cvec, rr], v)
            return 0
        lax.fori_loop(0, _HR // 4, tgroup, 0)

        def sgroup(g, _):
            sp = tr[s, _NBEV, pl.ds(g * 16, 16)]
            dq = dvrow[s, g // 8, pl.ds((g % 8) * 16, 16)]
            dvf = (dq & 15).astype(jnp.float32)
            seg = jnp.where(sp == 0.0, dvf, sp)
            tr[s, _NBEV, pl.ds(g * 16, 16)] = seg
            si = seg.astype(jnp.int32)
            for cls in range(16):
                oh[s, cls, pl.ds(g * 16, 16)] = (si == cls).astype(jnp.float32)
            return 0
        lax.fori_loop(0, _HR // 16, sgroup, 0)

    stage(0, 0)
    stage(1, 1)
    wait_stage(0)
    fire_gather(0, 0)

    def outer(j2, _):
        for u in range(2):
            def step(j=j2 * 2 + u, s=u):
                def nxt(j=j, s=s):
                    wait_stage(1 - s)
                    fire_gather(j + 1, 1 - s)
                pl.when(j + 1 < _NHRT)(nxt)
                wait_gather(s)

                def drain(j=j, s=s):
                    wait_outs(j - 2, s)
                pl.when(j >= 2)(drain)
                compute(j, s)

                def restage(j=j, s=s):
                    stage(j + 2, s)
                pl.when(j + 2 < _NHRT)(restage)
                fire_outs(j, s)
            pl.when(j2 * 2 + u < _NHRT)(step)
        return 0
    lax.fori_loop(0, _NHRT // 2, outer, 0)

    wait_outs(_NHRT - 2, 0)
    wait_outs(_NHRT - 1, 1)


def kernel(pillar_features, voxel_coords, pillar_seg_gt, pillar_dense_gt, dense_pillar_coords, points_mean):
    pm = points_mean.reshape(_P, 3)
    pf_ext = jnp.concatenate(
        [pillar_features, pillar_seg_gt, pm,
         jnp.zeros((_P, _PFW - 68), jnp.float32)], axis=1)
    pf_ext = jnp.concatenate([pf_ext, jnp.zeros((_NSENT, _PFW), jnp.float32)],
                             axis=0)

    pad = _NPC * _CH - _PB
    yp = jnp.pad(voxel_coords[:, 2], (0, pad + 8))
    xp = jnp.pad(voxel_coords[:, 3], (0, pad + 8))
    yd = dense_pillar_coords[:, 2]
    xd = dense_pillar_coords[:, 3]
    dval = pillar_dense_gt.reshape(-1)

    mesh = plsc.VectorSubcoreMesh(core_axis_name="c", subcore_axis_name="s")
    sc_params = pltpu.CompilerParams(
        needs_layout_passes=False, use_tc_tiling_on_sc=False)

    w_flat, dvp_flat = pl.kernel(
        _sc_scan,
        mesh=mesh,
        compiler_params=sc_params,
        out_type=[
            jax.ShapeDtypeStruct((_CELLS,), jnp.int32),
            jax.ShapeDtypeStruct((_CELLS,), jnp.int32),
        ],
        scratch_types=[
            pltpu.VMEM((_CPT,), jnp.int32),
            pltpu.VMEM((_CPT,), jnp.int32),
            pltpu.VMEM((2, _CH), jnp.int32),
            pltpu.VMEM((2, _CH), jnp.int32),
            pltpu.VMEM((2, _CH), jnp.float32),
            pltpu.SemaphoreType.DMA,
            pltpu.SemaphoreType.DMA,
        ],
    )(yp, xp, yd, xd, dval)

    w2 = w_flat.reshape(_CELLS // 128, 128)
    dvp2 = dvp_flat.reshape(_CELLS // 128, 128)

    feats2, seg2, pm2, oh2 = pl.kernel(
        _sc_produce,
        mesh=mesh,
        compiler_params=sc_params,
        out_type=[
            jax.ShapeDtypeStruct((_B * _NBEV, _NY * _NX), jnp.float32),
            jax.ShapeDtypeStruct((_B, _NY * _NX), jnp.float32),
            jax.ShapeDtypeStruct((_B * 3, _NY * _NX), jnp.float32),
            jax.ShapeDtypeStruct((_B * 16, _NY * _NX), jnp.float32),
        ],
        scratch_types=[
            pltpu.VMEM((2, 2, 128), jnp.int32),
            pltpu.VMEM((2, 2, 128), jnp.int32),
            pltpu.VMEM((2, _HR, _PFW), jnp.float32),
            pltpu.VMEM((2, 68, _TPITCH), jnp.float32),
            pltpu.VMEM((2, 16, _HR), jnp.float32),
            pltpu.SemaphoreType.DMA,
            pltpu.SemaphoreType.DMA,
            pltpu.SemaphoreType.DMA,
            pltpu.SemaphoreType.DMA,
            pltpu.SemaphoreType.DMA,
            pltpu.SemaphoreType.DMA,
        ],
    )(w2, dvp2, pf_ext)

    feats = feats2.reshape(_B, _NBEV, _NY, _NX)
    seg = seg2.reshape(_B, 1, _NY, _NX)
    pm3 = pm2.reshape(_B, 3, _NY, _NX)
    onehot = oh2.reshape(_B, 16, _NY, _NX)
    return feats, seg, pm3, onehot
